# baseline (speedup 1.0000x reference)
import jax
import jax.numpy as jnp
import numpy as np
from jax.experimental import pallas as pl
from jax.experimental.pallas import tpu as pltpu

_INV_EM1 = float(1.0 / (np.exp(1.0) - 1.0))
_L = 128
_K = 1024


def _vel_kernel(pos_ref, vel_ref, velout_ref, gamma_ref):
    x = pos_ref[:, 0, :]
    y = pos_ref[:, 1, :]
    vx = vel_ref[:, 0, :]
    vy = vel_ref[:, 1, :]
    m = (x >= 0.0) & (x <= 0.25) & (y >= 0.0) & (y <= 1.0)
    xr = jnp.clip(x * 4.0, 0.0, 1.0)
    t = xr * xr * xr * jnp.sqrt(xr)
    g = (jnp.exp(t) - 1.0) * _INV_EM1
    g = jnp.minimum(g, 1.0)
    velout_ref[:, 0, :] = jnp.where(m, vx + g * (1.0 - vx), vx)
    velout_ref[:, 1, :] = jnp.where(m, vy * (1.0 - g), vy)
    gamma_ref[...] = g.reshape(_K * _L)


def kernel(fluidPosition, fluidVelocity, fluidArea):
    n = fluidPosition.shape[0]
    nk = n // _L
    pos = fluidPosition.reshape(nk, _L, 2).swapaxes(1, 2)
    vel = fluidVelocity.reshape(nk, _L, 2).swapaxes(1, 2)
    grid = (nk + _K - 1) // _K
    vel_out, gamma = pl.pallas_call(
        _vel_kernel,
        grid=(grid,),
        in_specs=[
            pl.BlockSpec((_K, 2, _L), lambda i: (i, 0, 0)),
            pl.BlockSpec((_K, 2, _L), lambda i: (i, 0, 0)),
        ],
        out_specs=[
            pl.BlockSpec((_K, 2, _L), lambda i: (i, 0, 0)),
            pl.BlockSpec((_K * _L,), lambda i: (i,)),
        ],
        out_shape=[
            jax.ShapeDtypeStruct((nk, 2, _L), jnp.float32),
            jax.ShapeDtypeStruct((n,), jnp.float32),
        ],
        compiler_params=pltpu.CompilerParams(
            dimension_semantics=("parallel",),
        ),
    )(pos, vel)
    vel_out = vel_out.swapaxes(1, 2).reshape(n, 2)
    return vel_out, gamma


# speedup vs baseline: 1.0357x; 1.0357x over previous
import jax
import jax.numpy as jnp
import numpy as np
from jax.experimental import pallas as pl
from jax.experimental.pallas import tpu as pltpu

_INV_EM1 = float(1.0 / (np.exp(1.0) - 1.0))
_L = 128
_K = 512


def _vel_kernel(pos_ref, vel_ref, velout_ref, gamma_ref):
    x = pos_ref[:, 0, :]
    y = pos_ref[:, 1, :]
    vx = vel_ref[:, 0, :]
    vy = vel_ref[:, 1, :]
    m = (x >= 0.0) & (x <= 0.25) & (y >= 0.0) & (y <= 1.0)
    xr = jnp.clip(x * 4.0, 0.0, 1.0)
    t = xr * xr * xr * jnp.sqrt(xr)
    g = (jnp.exp(t) - 1.0) * _INV_EM1
    g = jnp.minimum(g, 1.0)
    velout_ref[:, 0, :] = jnp.where(m, vx + g * (1.0 - vx), vx)
    velout_ref[:, 1, :] = jnp.where(m, vy * (1.0 - g), vy)
    gamma_ref[...] = g.reshape(_K * _L)


def kernel(fluidPosition, fluidVelocity, fluidArea):
    n = fluidPosition.shape[0]
    nk = n // _L
    pos = fluidPosition.reshape(nk, _L, 2).swapaxes(1, 2)
    vel = fluidVelocity.reshape(nk, _L, 2).swapaxes(1, 2)
    grid = (nk + _K - 1) // _K
    vel_out, gamma = pl.pallas_call(
        _vel_kernel,
        grid=(grid,),
        in_specs=[
            pl.BlockSpec((_K, 2, _L), lambda i: (i, 0, 0)),
            pl.BlockSpec((_K, 2, _L), lambda i: (i, 0, 0)),
        ],
        out_specs=[
            pl.BlockSpec((_K, 2, _L), lambda i: (i, 0, 0)),
            pl.BlockSpec((_K * _L,), lambda i: (i,)),
        ],
        out_shape=[
            jax.ShapeDtypeStruct((nk, 2, _L), jnp.float32),
            jax.ShapeDtypeStruct((n,), jnp.float32),
        ],
        compiler_params=pltpu.CompilerParams(
            dimension_semantics=("parallel",),
        ),
    )(pos, vel)
    vel_out = vel_out.swapaxes(1, 2).reshape(n, 2)
    return vel_out, gamma


# speedup vs baseline: 1.0367x; 1.0010x over previous
import jax
import jax.numpy as jnp
import numpy as np
from jax.experimental import pallas as pl
from jax.experimental.pallas import tpu as pltpu

_INV_EM1 = float(1.0 / (np.exp(1.0) - 1.0))
_L = 128
_K = 512


def _vel_kernel(pos_ref, vel_ref, velout_ref, gamma_ref):
    x = pos_ref[:, 0, :]
    y = pos_ref[:, 1, :]
    vx = vel_ref[:, 0, :]
    vy = vel_ref[:, 1, :]
    m = (x >= 0.0) & (x <= 0.25) & (y >= 0.0) & (y <= 1.0)
    xr = jnp.clip(x * 4.0, 0.0, 1.0)
    t = xr * xr * xr * jnp.sqrt(xr)
    g = (jnp.exp(t) - 1.0) * _INV_EM1
    g = jnp.minimum(g, 1.0)
    velout_ref[:, 0, :] = jnp.where(m, vx + g * (1.0 - vx), vx)
    velout_ref[:, 1, :] = jnp.where(m, vy * (1.0 - g), vy)
    gamma_ref[...] = g.reshape(_K * _L)


def kernel(fluidPosition, fluidVelocity, fluidArea):
    n = fluidPosition.shape[0]
    nk = n // _L
    pos = fluidPosition.reshape(nk, _L, 2).swapaxes(1, 2)
    vel = fluidVelocity.reshape(nk, _L, 2).swapaxes(1, 2)
    grid = (nk + _K - 1) // _K
    vel_out, gamma = pl.pallas_call(
        _vel_kernel,
        grid=(grid,),
        in_specs=[
            pl.BlockSpec((_K, 2, _L), lambda i: (i, 0, 0)),
            pl.BlockSpec((_K, 2, _L), lambda i: (i, 0, 0)),
        ],
        out_specs=[
            pl.BlockSpec((_K, 2, _L), lambda i: (i, 0, 0)),
            pl.BlockSpec((_K * _L,), lambda i: (i,)),
        ],
        out_shape=[
            jax.ShapeDtypeStruct((nk, 2, _L), jnp.float32),
            jax.ShapeDtypeStruct((n,), jnp.float32),
        ],
    )(pos, vel)
    vel_out = vel_out.swapaxes(1, 2).reshape(n, 2)
    return vel_out, gamma


# speedup vs baseline: 1.1816x; 1.1398x over previous
import jax
import jax.numpy as jnp
import numpy as np
from jax.experimental import pallas as pl
from jax.experimental.pallas import tpu as pltpu

_INV_EM1 = float(1.0 / (np.exp(1.0) - 1.0))
_L = 128
_K = 512


def _vel_kernel(pos_ref, vel_ref, velout_ref, gamma_ref):
    x = pos_ref[:, 0, :]
    y = pos_ref[:, 1, :]
    vx = vel_ref[:, 0, :]
    vy = vel_ref[:, 1, :]
    m = (x >= 0.0) & (x <= 0.25) & (y >= 0.0) & (y <= 1.0)
    xr = jnp.clip(x * 4.0, 0.0, 1.0)
    t = jnp.exp(jnp.log(xr) * 3.5)
    g = (jnp.exp(t) - 1.0) * _INV_EM1
    g = jnp.minimum(g, 1.0)
    velout_ref[:, 0, :] = jnp.where(m, vx + g * (1.0 - vx), vx)
    velout_ref[:, 1, :] = jnp.where(m, vy * (1.0 - g), vy)
    gamma_ref[...] = g.reshape(_K * _L)


def kernel(fluidPosition, fluidVelocity, fluidArea):
    n = fluidPosition.shape[0]
    nk = n // _L
    pos = fluidPosition.reshape(nk, _L, 2).swapaxes(1, 2)
    vel = fluidVelocity.reshape(nk, _L, 2).swapaxes(1, 2)
    grid = (nk + _K - 1) // _K
    vel_out, gamma = pl.pallas_call(
        _vel_kernel,
        grid=(grid,),
        in_specs=[
            pl.BlockSpec((_K, 2, _L), lambda i: (i, 0, 0)),
            pl.BlockSpec((_K, 2, _L), lambda i: (i, 0, 0)),
        ],
        out_specs=[
            pl.BlockSpec((_K, 2, _L), lambda i: (i, 0, 0)),
            pl.BlockSpec((_K * _L,), lambda i: (i,)),
        ],
        out_shape=[
            jax.ShapeDtypeStruct((nk, 2, _L), jnp.float32),
            jax.ShapeDtypeStruct((n,), jnp.float32),
        ],
    )(pos, vel)
    vel_out = vel_out.swapaxes(1, 2).reshape(n, 2)
    return vel_out, gamma
